# baseline (speedup 1.0000x reference)
import numpy as np

import jax
import jax.numpy as jnp
from jax.experimental import pallas as pl
from jax.experimental.pallas import tpu as pltpu
from jax.sharding import Mesh, PartitionSpec as P

try:
    from jax import shard_map as _shard_map

    def _smap(f, mesh, in_specs, out_specs):
        return _shard_map(f, mesh=mesh, in_specs=in_specs,
                          out_specs=out_specs, check_vma=False)
except ImportError:
    from jax.experimental.shard_map import shard_map as _shard_map

    def _smap(f, mesh, in_specs, out_specs):
        return _shard_map(f, mesh=mesh, in_specs=in_specs,
                          out_specs=out_specs, check_rep=False)

K_BALL = 16
RADIUS2 = 0.2 * 0.2
TN = 256
TM = 512


def _body(pc_ref, mask_ref, pct_ref, maskt_ref, out_ref):
    b = pl.program_id(0)
    i = pl.program_id(1)

    pcb = pc_ref[0]
    pct = pct_ref[0]
    maskb = mask_ref[0].astype(jnp.bfloat16)
    maskt = maskt_ref[0].astype(jnp.bfloat16)

    n_total = pct.shape[1]

    rows = jax.lax.broadcasted_iota(jnp.int32, (TM, TM), 0)
    cols = jax.lax.broadcasted_iota(jnp.int32, (TM, TM), 1)
    tri = (rows < cols).astype(jnp.bfloat16)

    cnt = jnp.zeros((TN, 1), jnp.float32)
    acc = jnp.zeros((TN, TM), jnp.bfloat16)

    for c0 in range(0, n_total, TM):
        d2 = jnp.zeros((TN, TM), jnp.float32)
        for c in range(3):
            diff = pcb[:, c:c + 1] - pct[c:c + 1, c0:c0 + TM]
            d2 = d2 + diff * diff
        within = d2 < RADIUS2
        excl = jax.lax.dot_general(
            within.astype(jnp.bfloat16), tri,
            (((1,), (0,)), ((), ())),
            preferred_element_type=jnp.float32)
        sel = within & (cnt + excl < K_BALL)

        l1 = jnp.zeros((TN, TM), jnp.bfloat16)
        for c in range(16):
            l1 = l1 + jnp.abs(maskb[:, c:c + 1] - maskt[c:c + 1, c0:c0 + TM])

        acc = acc + jnp.where(sel, l1, jnp.bfloat16(0))
        cnt = cnt + excl[:, -1:] + within[:, -1:].astype(jnp.float32)

    @pl.when((b == 0) & (i == 0))
    def _init():
        out_ref[...] = jnp.zeros_like(out_ref)

    out_ref[...] += jnp.sum(acc.astype(jnp.float32)).reshape(1, 1)


def _per_shard(pc, mask):
    Bs, N, _ = pc.shape
    pct = jnp.transpose(pc, (0, 2, 1))
    maskt = jnp.transpose(mask, (0, 2, 1))
    total = pl.pallas_call(
        _body,
        grid=(Bs, N // TN),
        in_specs=[
            pl.BlockSpec((1, TN, 3), lambda b, i: (b, i, 0)),
            pl.BlockSpec((1, TN, 16), lambda b, i: (b, i, 0)),
            pl.BlockSpec((1, 3, N), lambda b, i: (b, 0, 0)),
            pl.BlockSpec((1, 16, N), lambda b, i: (b, 0, 0)),
        ],
        out_specs=pl.BlockSpec((1, 1), lambda b, i: (0, 0)),
        out_shape=jax.ShapeDtypeStruct((1, 1), jnp.float32),
        compiler_params=pltpu.CompilerParams(
            dimension_semantics=("arbitrary", "arbitrary")),
    )(pc, mask, pct, maskt)
    return jax.lax.psum(total, "d")


def kernel(pc, mask):
    B, N, _ = pc.shape
    devs = jax.devices()
    nd = max(d for d in range(1, min(len(devs), B) + 1) if B % d == 0)
    mesh = Mesh(np.asarray(devs[:nd]), ("d",))
    total = _smap(_per_shard, mesh,
                  (P("d"), P("d")), P())(pc, mask)
    return total[0, 0] / (B * N * K_BALL)


# speedup vs baseline: 2.8499x; 2.8499x over previous
import jax
import jax.numpy as jnp
from jax.experimental import pallas as pl
from jax.experimental.pallas import tpu as pltpu

K_BALL = 16
RADIUS2 = 0.2 * 0.2
TN = 256
TM = 512


def _body(pc_ref, mask_ref, pct_ref, maskt_ref, out_ref):
    b = pl.program_id(0)
    i = pl.program_id(1)

    pcb = pc_ref[0]
    pct = pct_ref[0]
    maskb = mask_ref[0].astype(jnp.bfloat16)
    maskt = maskt_ref[0].astype(jnp.bfloat16)

    n_total = pct.shape[1]

    rows = jax.lax.broadcasted_iota(jnp.int32, (TM, TM), 0)
    cols = jax.lax.broadcasted_iota(jnp.int32, (TM, TM), 1)
    tri = (rows < cols).astype(jnp.bfloat16)

    cnt = jnp.zeros((TN, 1), jnp.float32)
    acc = jnp.zeros((TN, TM), jnp.bfloat16)

    for c0 in range(0, n_total, TM):
        d2 = jnp.zeros((TN, TM), jnp.float32)
        for c in range(3):
            diff = pcb[:, c:c + 1] - pct[c:c + 1, c0:c0 + TM]
            d2 = d2 + diff * diff
        within = d2 < RADIUS2
        excl = jax.lax.dot_general(
            within.astype(jnp.bfloat16), tri,
            (((1,), (0,)), ((), ())),
            preferred_element_type=jnp.float32)
        sel = within & (cnt + excl < K_BALL)

        l1 = jnp.zeros((TN, TM), jnp.bfloat16)
        for c in range(16):
            l1 = l1 + jnp.abs(maskb[:, c:c + 1] - maskt[c:c + 1, c0:c0 + TM])

        acc = acc + jnp.where(sel, l1, jnp.bfloat16(0))
        cnt = cnt + excl[:, -1:] + within[:, -1:].astype(jnp.float32)

    @pl.when((b == 0) & (i == 0))
    def _init():
        out_ref[...] = jnp.zeros_like(out_ref)

    out_ref[...] += jnp.sum(acc.astype(jnp.float32)).reshape(1, 1)


def _run(pc, mask):
    Bs, N, _ = pc.shape
    pct = jnp.transpose(pc, (0, 2, 1))
    maskt = jnp.transpose(mask, (0, 2, 1))
    total = pl.pallas_call(
        _body,
        grid=(Bs, N // TN),
        in_specs=[
            pl.BlockSpec((1, TN, 3), lambda b, i: (b, i, 0)),
            pl.BlockSpec((1, TN, 16), lambda b, i: (b, i, 0)),
            pl.BlockSpec((1, 3, N), lambda b, i: (b, 0, 0)),
            pl.BlockSpec((1, 16, N), lambda b, i: (b, 0, 0)),
        ],
        out_specs=pl.BlockSpec((1, 1), lambda b, i: (0, 0)),
        out_shape=jax.ShapeDtypeStruct((1, 1), jnp.float32),
        compiler_params=pltpu.CompilerParams(
            dimension_semantics=("arbitrary", "arbitrary")),
    )(pc, mask, pct, maskt)
    return total


def kernel(pc, mask):
    B, N, _ = pc.shape
    total = _run(pc, mask)
    return total[0, 0] / (B * N * K_BALL)


# speedup vs baseline: 2.9040x; 1.0190x over previous
import jax
import jax.numpy as jnp
from jax.experimental import pallas as pl
from jax.experimental.pallas import tpu as pltpu

K_BALL = 16
RADIUS2 = 0.2 * 0.2
TN = 512
TM = 512


def _body(pc_ref, mask_ref, pct_ref, maskt_ref, tri_ref, out_ref):
    b = pl.program_id(0)
    i = pl.program_id(1)

    pcb = pc_ref[0]
    pct = pct_ref[0]
    maskb = mask_ref[0].astype(jnp.bfloat16)
    maskt = maskt_ref[0].astype(jnp.bfloat16)
    tri = tri_ref[...]

    n_total = pct.shape[1]

    cnt = jnp.zeros((TN, 1), jnp.float32)
    acc = jnp.zeros((TN, TM), jnp.bfloat16)

    for c0 in range(0, n_total, TM):
        d2 = jnp.zeros((TN, TM), jnp.float32)
        for c in range(3):
            diff = pcb[:, c:c + 1] - pct[c:c + 1, c0:c0 + TM]
            d2 = d2 + diff * diff
        within = d2 < RADIUS2
        excl = jax.lax.dot_general(
            within.astype(jnp.bfloat16), tri,
            (((1,), (0,)), ((), ())),
            preferred_element_type=jnp.float32)
        sel = within & (cnt + excl < K_BALL)

        l1 = jnp.zeros((TN, TM), jnp.bfloat16)
        for c in range(16):
            l1 = l1 + jnp.abs(maskb[:, c:c + 1] - maskt[c:c + 1, c0:c0 + TM])

        acc = acc + jnp.where(sel, l1, jnp.bfloat16(0))
        cnt = cnt + excl[:, -1:] + within[:, -1:].astype(jnp.float32)

    @pl.when((b == 0) & (i == 0))
    def _init():
        out_ref[...] = jnp.zeros_like(out_ref)

    out_ref[...] += jnp.sum(acc.astype(jnp.float32)).reshape(1, 1)


def _run(pc, mask):
    Bs, N, _ = pc.shape
    pct = jnp.transpose(pc, (0, 2, 1))
    maskt = jnp.transpose(mask, (0, 2, 1))
    tri = (jnp.arange(TM)[:, None] < jnp.arange(TM)[None, :]
           ).astype(jnp.bfloat16)
    total = pl.pallas_call(
        _body,
        grid=(Bs, N // TN),
        in_specs=[
            pl.BlockSpec((1, TN, 3), lambda b, i: (b, i, 0)),
            pl.BlockSpec((1, TN, 16), lambda b, i: (b, i, 0)),
            pl.BlockSpec((1, 3, N), lambda b, i: (b, 0, 0)),
            pl.BlockSpec((1, 16, N), lambda b, i: (b, 0, 0)),
            pl.BlockSpec((TM, TM), lambda b, i: (0, 0)),
        ],
        out_specs=pl.BlockSpec((1, 1), lambda b, i: (0, 0)),
        out_shape=jax.ShapeDtypeStruct((1, 1), jnp.float32),
        compiler_params=pltpu.CompilerParams(
            dimension_semantics=("arbitrary", "arbitrary")),
    )(pc, mask, pct, maskt, tri)
    return total


def kernel(pc, mask):
    B, N, _ = pc.shape
    total = _run(pc, mask)
    return total[0, 0] / (B * N * K_BALL)
